# baseline (speedup 1.0000x reference)
import jax
import jax.numpy as jnp
from jax import lax
from jax.experimental import pallas as pl
from jax.experimental.pallas import tpu as pltpu
from jax.experimental.pallas import tpu_sc as plsc

N = 50000
D = 128
R = 4
E = 150000
L = 2 * R

C = 1200
CHUNKS = 128
EP = CHUNKS * C
CPT = CHUNKS // 16
NQ = 4
Q = 12544
QP = 12608
RPT = Q // 16
G = 48
NBMAX = (C + G - 1) // G
CPAD = NBMAX * G + G
NSLOT = 2
CP = CPAD
HB = Q // 2
MB = 2000


def _mm_body(x_ref, w_ref, b_ref, y_ref):
    y_ref[0] = (
        jnp.dot(x_ref[...], w_ref[0], preferred_element_type=jnp.float32)
        + b_ref[0]
    )


def _mm_stage(x, w_all, b_all):
    return pl.pallas_call(
        _mm_body,
        grid=(N // MB, L),
        in_specs=[
            pl.BlockSpec((MB, D), lambda nb, l: (nb, 0)),
            pl.BlockSpec((1, D, D), lambda nb, l: (l, 0, 0)),
            pl.BlockSpec((1, 1, D), lambda nb, l: (l, 0, 0)),
        ],
        out_specs=pl.BlockSpec((1, MB, D), lambda nb, l: (l, nb, 0)),
        out_shape=jax.ShapeDtypeStruct((L, N, D), jnp.float32),
    )(x, w_all, b_all.reshape(L, 1, D))


def _norm_body(o_ref, d_ref, ones_ref, out_ref):
    degb = lax.dot_general(
        d_ref[0], ones_ref[...], (((0,), (0,)), ((), ())),
        preferred_element_type=jnp.float32,
    )
    inv = 1.0 / jnp.where(degb == 0.0, 1.0, degb)
    out_ref[...] = o_ref[...] * inv


def _norm_stage(out_u, deg):
    ones = jnp.ones((1, D), jnp.float32)
    return pl.pallas_call(
        _norm_body,
        grid=(NQ * 2,),
        in_specs=[
            pl.BlockSpec((HB, D), lambda nb: (nb, 0)),
            pl.BlockSpec((1, 1, HB), lambda nb: (nb, 0, 0)),
            pl.BlockSpec((1, D), lambda nb: (0, 0)),
        ],
        out_specs=pl.BlockSpec((HB, D), lambda nb: (nb, 0)),
        out_shape=jax.ShapeDtypeStruct((N, D), jnp.float32),
    )(out_u, deg.reshape(NQ * 2, 1, HB), ones)


def _sc_body(y, rows, cols, z2d,
             out_u, deg,
             acc_sp, deg_sp, zidx, zv, dv, ridx, cidx, fridx, fcidx, sidx,
             gbuf, ones_g, sem, semg, sems, semi):
    c = lax.axis_index("c")
    t = lax.axis_index("s")
    for j in range(G // 16):
        ones_g[pl.ds(16 * j, 16)] = jnp.ones((16,), jnp.float32)
    t0 = pl.multiple_of(lax.axis_index("s") * RPT, 16)

    def zfill(i, _):
        zidx[pl.ds(16 * i, 16)] = t0 + 16 * i + lax.iota(jnp.int32, 16)
        zv[pl.ds(16 * i, 16)] = jnp.zeros((16,), jnp.float32)
        return 0
    lax.fori_loop(0, RPT // 16, zfill, 0)

    for jj in range(2):
        s = 2 * c + jj
        lo = s * Q
        tr0 = pl.multiple_of(t * RPT, 16)
        pltpu.sync_copy(z2d, acc_sp.at[pl.ds(tr0, RPT)])
        pltpu.sync_copy(zv, deg_sp.at[zidx])
        plsc.subcore_barrier()

        def idx_base(cc):
            l = cc // CPT
            k = cc - l * CPT
            return l, pl.multiple_of(l * EP + (t * CPT + k) * C, 128)

        def fire_idx(cc, islot):
            _l, base = idx_base(cc)
            ro = pl.multiple_of(islot * C, 64)
            pltpu.async_copy(rows.at[pl.ds(base, C)], ridx.at[pl.ds(ro, C)],
                             semi.at[islot])
            pltpu.async_copy(cols.at[pl.ds(base, C)], cidx.at[pl.ds(ro, C)],
                             semi.at[islot])

        fire_idx(0, 0)

        if True:
            def chunk_body(cc, _):
                l, base = idx_base(cc)
                islot = cc % 2
                @pl.when(cc + 1 < L * CPT)
                def _():
                    fire_idx(cc + 1, 1 - islot)
                ro = pl.multiple_of(islot * C, 64)
                pltpu.make_async_copy(rows.at[pl.ds(0, C)],
                                     ridx.at[pl.ds(ro, C)],
                                     semi.at[islot]).wait()
                pltpu.make_async_copy(cols.at[pl.ds(0, C)],
                                     cidx.at[pl.ds(ro, C)],
                                     semi.at[islot]).wait()
                rid = ridx.at[pl.ds(ro, C)]
                cid = cidx.at[pl.ds(ro, C)]

                def fvreg(i, p):
                    rv = rid[pl.ds(16 * i, 16)]
                    cv = cid[pl.ds(16 * i, 16)]
                    loc = rv - lo
                    m = (loc >= 0) & (loc < Q)
                    mi = jnp.where(m, jnp.int32(1), jnp.int32(0))
                    cs = plsc.cumsum(mi)
                    pos = p + cs - mi
                    plsc.store_scatter(fridx, [pos], loc, mask=m)
                    plsc.store_scatter(fcidx, [pos], cv, mask=m)
                    return p + cs[15]

                p = lax.fori_loop(0, C // 16, fvreg, 0)
                iota = lax.iota(jnp.int32, 16)
                for j in range(G // 16):
                    fridx[pl.ds(p + 16 * j, 16)] = Q + 16 * j + iota
                    fcidx[pl.ds(p + 16 * j, 16)] = iota * 3000 + 750 * j + 7
                nb = (p + G - 1) // G

                def sidx_fill(b, _):
                    o = pl.multiple_of(b * G, G)
                    srow = sidx.at[b]
                    for j in range(G // 16):
                        srow[pl.ds(16 * j, 16)] = fridx[pl.ds(o + 16 * j, 16)]
                    return 0
                lax.fori_loop(0, nb, sidx_fill, 0)

                def fire_gather(b, slot):
                    o = pl.multiple_of(b * G, G)
                    pltpu.async_copy(
                        y.at[l].at[fcidx.at[pl.ds(o, G)]],
                        gbuf.at[slot], semg.at[slot],
                    )

                for pre in range(NSLOT):
                    @pl.when(pre < nb)
                    def _():
                        fire_gather(pre, pre)

                def batch_body(b, _):
                    slot = b % NSLOT
                    gslot = gbuf.at[slot]
                    pltpu.make_async_copy(
                        y.at[l].at[fcidx.at[pl.ds(0, G)]], gslot,
                        semg.at[slot]).wait()
                    pltpu.async_copy(gslot, acc_sp.at[sidx.at[b]],
                                     sems.at[slot], add=True)
                    pltpu.async_copy(ones_g, deg_sp.at[sidx.at[b]],
                                     sems.at[slot], add=True)
                    @pl.when(b + NSLOT < nb)
                    def _():
                        pltpu.make_async_copy(
                            gslot, acc_sp.at[sidx.at[b]],
                            sems.at[slot]).wait()
                        pltpu.make_async_copy(
                            ones_g, deg_sp.at[sidx.at[b]],
                            sems.at[slot]).wait()
                        fire_gather(b + NSLOT, slot)
                    return 0
                lax.fori_loop(0, nb, batch_body, 0)

                def tail_drain(e, _):
                    @pl.when(e < nb)
                    def _():
                        slot = e % NSLOT
                        pltpu.make_async_copy(
                            gbuf.at[slot], acc_sp.at[sidx.at[e]],
                            sems.at[slot]).wait()
                        pltpu.make_async_copy(
                            ones_g, deg_sp.at[sidx.at[e]],
                            sems.at[slot]).wait()
                    return 0
                nb0 = jnp.minimum(nb, NSLOT)
                start = jnp.maximum(nb - NSLOT, 0)
                lax.fori_loop(start, start + nb0, tail_drain, 0)
                return 0
            lax.fori_loop(0, L * CPT, chunk_body, 0)

        plsc.subcore_barrier()
        w0 = pl.multiple_of(s * Q + tr0, 16)
        last = (s == NQ - 1) & (t == 15)
        @pl.when(~last)
        def _():
            pltpu.sync_copy(acc_sp.at[pl.ds(tr0, RPT)],
                            out_u.at[pl.ds(w0, RPT)])
        @pl.when(last)
        def _():
            nl = N - (NQ - 1) * Q - 15 * RPT
            pltpu.sync_copy(acc_sp.at[pl.ds(tr0, nl)],
                            out_u.at[pl.ds(w0, nl)])
        pltpu.async_copy(deg_sp.at[zidx], dv, sem).wait()
        pltpu.sync_copy(dv, deg.at[pl.ds(w0, RPT)])
        plsc.subcore_barrier()


def _sc_stage(y, rows, cols):
    z2d = jnp.zeros((RPT, D), jnp.float32)
    f = pl.kernel(
        _sc_body,
        out_type=[
            jax.ShapeDtypeStruct((N, D), jnp.float32),
            jax.ShapeDtypeStruct((NQ * Q,), jnp.float32),
        ],
        mesh=plsc.VectorSubcoreMesh(core_axis_name="c", subcore_axis_name="s"),
        scratch_types=[
            pltpu.VMEM_SHARED((QP, D), jnp.float32),
            pltpu.VMEM_SHARED((QP,), jnp.float32),
            pltpu.VMEM((RPT,), jnp.int32),
            pltpu.VMEM((RPT,), jnp.float32),
            pltpu.VMEM((RPT,), jnp.float32),
            pltpu.VMEM((2 * C,), jnp.int32),
            pltpu.VMEM((2 * C,), jnp.int32),
            pltpu.VMEM((CP,), jnp.int32),
            pltpu.VMEM((CP,), jnp.int32),
            pltpu.VMEM((NBMAX, G), jnp.int32),
            pltpu.VMEM((NSLOT, G, D), jnp.float32),
            pltpu.VMEM((G,), jnp.float32),
            pltpu.SemaphoreType.DMA,
            pltpu.SemaphoreType.DMA((NSLOT,)),
            pltpu.SemaphoreType.DMA((NSLOT,)),
            pltpu.SemaphoreType.DMA((2,)),
        ],
        compiler_params=pltpu.CompilerParams(needs_layout_passes=False),
    )
    return f(y, rows, cols, z2d)


def kernel(features, adjacencies, adjacencies_t, w, bias, w_t, bias_t):
    w_all = jnp.concatenate([w, w_t], axis=0)
    b_all = jnp.concatenate([bias, bias_t], axis=0)

    pad_r = jnp.broadcast_to(jnp.int32(2 * N), (L, EP - E))
    pad_c = jnp.broadcast_to(
        (jnp.arange(EP - E, dtype=jnp.int32) * 977) % N, (L, EP - E))
    rows = jnp.concatenate(
        [adjacencies[:, 0, :], adjacencies_t[:, 0, :]], axis=0)
    cols = jnp.concatenate(
        [adjacencies[:, 1, :], adjacencies_t[:, 1, :]], axis=0)
    rows_p = jnp.concatenate([rows, pad_r], axis=1).reshape(L * EP)
    cols_p = jnp.concatenate([cols, pad_c], axis=1).reshape(L * EP)

    y = _mm_stage(features, w_all, b_all)
    out_u, deg = _sc_stage(y, rows_p, cols_p)
    return _norm_stage(out_u, deg)


# speedup vs baseline: 1.1166x; 1.1166x over previous
import jax
import jax.numpy as jnp
from jax import lax
from jax.experimental import pallas as pl
from jax.experimental.pallas import tpu as pltpu
from jax.experimental.pallas import tpu_sc as plsc

N = 50000
D = 128
R = 4
E = 150000
L = 2 * R

C = 960
CHUNKS = 160
EP = CHUNKS * C
CPT = CHUNKS // 16
NQ = 4
Q = 12544
QP = 12576
RPT = Q // 16
G = 32
NBMAX = (C + G - 1) // G
CPAD = NBMAX * G + G
NSLOT = 4
CP = CPAD
HB = Q // 2
MB = 2000


def _mm_body(x_ref, w_ref, b_ref, y_ref):
    y_ref[0] = (
        jnp.dot(x_ref[...], w_ref[0], preferred_element_type=jnp.float32)
        + b_ref[0]
    )


def _mm_stage(x, w_all, b_all):
    return pl.pallas_call(
        _mm_body,
        grid=(N // MB, L),
        in_specs=[
            pl.BlockSpec((MB, D), lambda nb, l: (nb, 0)),
            pl.BlockSpec((1, D, D), lambda nb, l: (l, 0, 0)),
            pl.BlockSpec((1, 1, D), lambda nb, l: (l, 0, 0)),
        ],
        out_specs=pl.BlockSpec((1, MB, D), lambda nb, l: (l, nb, 0)),
        out_shape=jax.ShapeDtypeStruct((L, N, D), jnp.float32),
    )(x, w_all, b_all.reshape(L, 1, D))


def _norm_body(o_ref, d_ref, ones_ref, out_ref):
    degb = lax.dot_general(
        d_ref[0], ones_ref[...], (((0,), (0,)), ((), ())),
        preferred_element_type=jnp.float32,
    )
    inv = 1.0 / jnp.where(degb == 0.0, 1.0, degb)
    out_ref[...] = o_ref[...] * inv


def _norm_stage(out_u, deg):
    ones = jnp.ones((1, D), jnp.float32)
    return pl.pallas_call(
        _norm_body,
        grid=(NQ * 2,),
        in_specs=[
            pl.BlockSpec((HB, D), lambda nb: (nb, 0)),
            pl.BlockSpec((1, 1, HB), lambda nb: (nb, 0, 0)),
            pl.BlockSpec((1, D), lambda nb: (0, 0)),
        ],
        out_specs=pl.BlockSpec((HB, D), lambda nb: (nb, 0)),
        out_shape=jax.ShapeDtypeStruct((N, D), jnp.float32),
    )(out_u, deg.reshape(NQ * 2, 1, HB), ones)


def _sc_body(y, rows, cols, z2d,
             out_u, deg,
             acc_sp, deg_sp, zidx, zv, dv, ridx, cidx, fridx, fcidx, sidx,
             gbuf, ones_g, sem, semg, sems, semi):
    c = lax.axis_index("c")
    t = lax.axis_index("s")
    for j in range(G // 16):
        ones_g[pl.ds(16 * j, 16)] = jnp.ones((16,), jnp.float32)
    t0 = pl.multiple_of(lax.axis_index("s") * RPT, 16)

    def zfill(i, _):
        zidx[pl.ds(16 * i, 16)] = t0 + 16 * i + lax.iota(jnp.int32, 16)
        zv[pl.ds(16 * i, 16)] = jnp.zeros((16,), jnp.float32)
        return 0
    lax.fori_loop(0, RPT // 16, zfill, 0)

    for jj in range(2):
        s = 2 * c + jj
        lo = s * Q
        tr0 = pl.multiple_of(t * RPT, 16)
        pltpu.sync_copy(z2d, acc_sp.at[pl.ds(tr0, RPT)])
        pltpu.sync_copy(zv, deg_sp.at[zidx])
        plsc.subcore_barrier()

        def idx_base(cc):
            l = cc // CPT
            k = cc - l * CPT
            return l, pl.multiple_of(l * EP + (t * CPT + k) * C, 128)

        def fire_idx(cc, islot):
            _l, base = idx_base(cc)
            ro = pl.multiple_of(islot * C, 64)
            pltpu.async_copy(rows.at[pl.ds(base, C)], ridx.at[pl.ds(ro, C)],
                             semi.at[islot])
            pltpu.async_copy(cols.at[pl.ds(base, C)], cidx.at[pl.ds(ro, C)],
                             semi.at[islot])

        fire_idx(0, 0)

        if True:
            def chunk_body(cc, _):
                l, base = idx_base(cc)
                islot = cc % 2
                @pl.when(cc + 1 < L * CPT)
                def _():
                    fire_idx(cc + 1, 1 - islot)
                ro = pl.multiple_of(islot * C, 64)
                pltpu.make_async_copy(rows.at[pl.ds(0, C)],
                                     ridx.at[pl.ds(ro, C)],
                                     semi.at[islot]).wait()
                pltpu.make_async_copy(cols.at[pl.ds(0, C)],
                                     cidx.at[pl.ds(ro, C)],
                                     semi.at[islot]).wait()
                rid = ridx.at[pl.ds(ro, C)]
                cid = cidx.at[pl.ds(ro, C)]

                def fvreg(i, p):
                    rv = rid[pl.ds(16 * i, 16)]
                    cv = cid[pl.ds(16 * i, 16)]
                    loc = rv - lo
                    m = (loc >= 0) & (loc < Q)
                    mi = jnp.where(m, jnp.int32(1), jnp.int32(0))
                    cs = plsc.cumsum(mi)
                    pos = p + cs - mi
                    plsc.store_scatter(fridx, [pos], loc, mask=m)
                    plsc.store_scatter(fcidx, [pos], cv, mask=m)
                    return p + cs[15]

                p = lax.fori_loop(0, C // 16, fvreg, 0)
                iota = lax.iota(jnp.int32, 16)
                for j in range(G // 16):
                    fridx[pl.ds(p + 16 * j, 16)] = Q + 16 * j + iota
                    fcidx[pl.ds(p + 16 * j, 16)] = iota * 3000 + 750 * j + 7
                nb = (p + G - 1) // G

                def sidx_fill(b, _):
                    o = pl.multiple_of(b * G, G)
                    srow = sidx.at[b]
                    for j in range(G // 16):
                        srow[pl.ds(16 * j, 16)] = fridx[pl.ds(o + 16 * j, 16)]
                    return 0
                lax.fori_loop(0, nb, sidx_fill, 0)

                def fire_gather(b, slot):
                    o = pl.multiple_of(b * G, G)
                    pltpu.async_copy(
                        y.at[l].at[fcidx.at[pl.ds(o, G)]],
                        gbuf.at[slot], semg.at[slot],
                    )

                for pre in range(NSLOT):
                    @pl.when(pre < nb)
                    def _():
                        fire_gather(pre, pre)

                def batch_body(b, _):
                    slot = b % NSLOT
                    gslot = gbuf.at[slot]
                    pltpu.make_async_copy(
                        y.at[l].at[fcidx.at[pl.ds(0, G)]], gslot,
                        semg.at[slot]).wait()
                    pltpu.async_copy(gslot, acc_sp.at[sidx.at[b]],
                                     sems.at[slot], add=True)
                    pltpu.async_copy(ones_g, deg_sp.at[sidx.at[b]],
                                     sems.at[slot], add=True)
                    @pl.when(b + NSLOT < nb)
                    def _():
                        pltpu.make_async_copy(
                            gslot, acc_sp.at[sidx.at[b]],
                            sems.at[slot]).wait()
                        pltpu.make_async_copy(
                            ones_g, deg_sp.at[sidx.at[b]],
                            sems.at[slot]).wait()
                        fire_gather(b + NSLOT, slot)
                    return 0
                lax.fori_loop(0, nb, batch_body, 0)

                def tail_drain(e, _):
                    @pl.when(e < nb)
                    def _():
                        slot = e % NSLOT
                        pltpu.make_async_copy(
                            gbuf.at[slot], acc_sp.at[sidx.at[e]],
                            sems.at[slot]).wait()
                        pltpu.make_async_copy(
                            ones_g, deg_sp.at[sidx.at[e]],
                            sems.at[slot]).wait()
                    return 0
                nb0 = jnp.minimum(nb, NSLOT)
                start = jnp.maximum(nb - NSLOT, 0)
                lax.fori_loop(start, start + nb0, tail_drain, 0)
                return 0
            lax.fori_loop(0, L * CPT, chunk_body, 0)

        plsc.subcore_barrier()
        w0 = pl.multiple_of(s * Q + tr0, 16)
        last = (s == NQ - 1) & (t == 15)
        @pl.when(~last)
        def _():
            pltpu.sync_copy(acc_sp.at[pl.ds(tr0, RPT)],
                            out_u.at[pl.ds(w0, RPT)])
        @pl.when(last)
        def _():
            nl = N - (NQ - 1) * Q - 15 * RPT
            pltpu.sync_copy(acc_sp.at[pl.ds(tr0, nl)],
                            out_u.at[pl.ds(w0, nl)])
        pltpu.async_copy(deg_sp.at[zidx], dv, sem).wait()
        pltpu.sync_copy(dv, deg.at[pl.ds(w0, RPT)])
        plsc.subcore_barrier()


def _sc_stage(y, rows, cols):
    z2d = jnp.zeros((RPT, D), jnp.float32)
    f = pl.kernel(
        _sc_body,
        out_type=[
            jax.ShapeDtypeStruct((N, D), jnp.float32),
            jax.ShapeDtypeStruct((NQ * Q,), jnp.float32),
        ],
        mesh=plsc.VectorSubcoreMesh(core_axis_name="c", subcore_axis_name="s"),
        scratch_types=[
            pltpu.VMEM_SHARED((QP, D), jnp.float32),
            pltpu.VMEM_SHARED((QP,), jnp.float32),
            pltpu.VMEM((RPT,), jnp.int32),
            pltpu.VMEM((RPT,), jnp.float32),
            pltpu.VMEM((RPT,), jnp.float32),
            pltpu.VMEM((2 * C,), jnp.int32),
            pltpu.VMEM((2 * C,), jnp.int32),
            pltpu.VMEM((CP,), jnp.int32),
            pltpu.VMEM((CP,), jnp.int32),
            pltpu.VMEM((NBMAX, G), jnp.int32),
            pltpu.VMEM((NSLOT, G, D), jnp.float32),
            pltpu.VMEM((G,), jnp.float32),
            pltpu.SemaphoreType.DMA,
            pltpu.SemaphoreType.DMA((NSLOT,)),
            pltpu.SemaphoreType.DMA((NSLOT,)),
            pltpu.SemaphoreType.DMA((2,)),
        ],
        compiler_params=pltpu.CompilerParams(needs_layout_passes=False),
    )
    return f(y, rows, cols, z2d)


def kernel(features, adjacencies, adjacencies_t, w, bias, w_t, bias_t):
    w_all = jnp.concatenate([w, w_t], axis=0)
    b_all = jnp.concatenate([bias, bias_t], axis=0)

    pad_r = jnp.broadcast_to(jnp.int32(2 * N), (L, EP - E))
    pad_c = jnp.broadcast_to(
        (jnp.arange(EP - E, dtype=jnp.int32) * 977) % N, (L, EP - E))
    rows = jnp.concatenate(
        [adjacencies[:, 0, :], adjacencies_t[:, 0, :]], axis=0)
    cols = jnp.concatenate(
        [adjacencies[:, 1, :], adjacencies_t[:, 1, :]], axis=0)
    rows_p = jnp.concatenate([rows, pad_r], axis=1).reshape(L * EP)
    cols_p = jnp.concatenate([cols, pad_c], axis=1).reshape(L * EP)

    y = _mm_stage(features, w_all, b_all)
    out_u, deg = _sc_stage(y, rows_p, cols_p)
    return _norm_stage(out_u, deg)


# speedup vs baseline: 1.1604x; 1.0392x over previous
import jax
import jax.numpy as jnp
from jax import lax
from jax.experimental import pallas as pl
from jax.experimental.pallas import tpu as pltpu
from jax.experimental.pallas import tpu_sc as plsc

N = 50000
D = 128
R = 4
E = 150000
L = 2 * R

C = 960
CHUNKS = 160
EP = CHUNKS * C
CPT = CHUNKS // 16
NQ = 4
Q = 12544
QP = 12576
RPT = Q // 16
G = 32
NBMAX = (C + G - 1) // G
CPAD = NBMAX * G + G
NSLOT = 4
CP = CPAD
HB = Q // 2
MB = 2000


def _mm_body(x_ref, w_ref, b_ref, y_ref):
    y_ref[0] = (
        jnp.dot(x_ref[...], w_ref[0], preferred_element_type=jnp.float32)
        + b_ref[0]
    )


def _mm_stage(x, w_all, b_all):
    return pl.pallas_call(
        _mm_body,
        grid=(N // MB, L),
        in_specs=[
            pl.BlockSpec((MB, D), lambda nb, l: (nb, 0)),
            pl.BlockSpec((1, D, D), lambda nb, l: (l, 0, 0)),
            pl.BlockSpec((1, 1, D), lambda nb, l: (l, 0, 0)),
        ],
        out_specs=pl.BlockSpec((1, MB, D), lambda nb, l: (l, nb, 0)),
        out_shape=jax.ShapeDtypeStruct((L, N, D), jnp.float32),
    )(x, w_all, b_all.reshape(L, 1, D))


def _norm_body(o_ref, d_ref, ones_ref, out_ref):
    degb = lax.dot_general(
        d_ref[0], ones_ref[...], (((0,), (0,)), ((), ())),
        preferred_element_type=jnp.float32,
    )
    inv = 1.0 / jnp.where(degb == 0.0, 1.0, degb)
    out_ref[...] = o_ref[...] * inv


def _norm_stage(out_u, deg):
    ones = jnp.ones((1, D), jnp.float32)
    return pl.pallas_call(
        _norm_body,
        grid=(NQ * 2,),
        in_specs=[
            pl.BlockSpec((HB, D), lambda nb: (nb, 0)),
            pl.BlockSpec((1, 1, HB), lambda nb: (nb, 0, 0)),
            pl.BlockSpec((1, D), lambda nb: (0, 0)),
        ],
        out_specs=pl.BlockSpec((HB, D), lambda nb: (nb, 0)),
        out_shape=jax.ShapeDtypeStruct((N, D), jnp.float32),
    )(out_u, deg.reshape(NQ * 2, 1, HB), ones)


def _sc_body(y, rows, cols, z2d,
             out_u, deg,
             acc_sp, deg_sp, zidx, zv, dv, ridx, cidx, fridx, fcidx, sidx,
             gbuf, ones_g, sem, semg, sems, semi):
    c = lax.axis_index("c")
    t = lax.axis_index("s")
    for j in range(G // 16):
        ones_g[pl.ds(16 * j, 16)] = jnp.ones((16,), jnp.float32)
    t0 = pl.multiple_of(lax.axis_index("s") * RPT, 16)

    def zfill(i, _):
        zidx[pl.ds(16 * i, 16)] = t0 + 16 * i + lax.iota(jnp.int32, 16)
        zv[pl.ds(16 * i, 16)] = jnp.zeros((16,), jnp.float32)
        return 0
    lax.fori_loop(0, RPT // 16, zfill, 0)

    for jj in range(2):
        s = 2 * c + jj
        lo = s * Q
        tr0 = pl.multiple_of(t * RPT, 16)
        pltpu.sync_copy(z2d, acc_sp.at[pl.ds(tr0, RPT)])
        pltpu.sync_copy(zv, deg_sp.at[zidx])
        plsc.subcore_barrier()

        def idx_base(cc):
            l = cc // CPT
            k = cc - l * CPT
            return l, pl.multiple_of(l * EP + (t * CPT + k) * C, 128)

        def fire_idx(cc, islot):
            _l, base = idx_base(cc)
            ro = pl.multiple_of(islot * C, 64)
            pltpu.async_copy(rows.at[pl.ds(base, C)], ridx.at[pl.ds(ro, C)],
                             semi.at[islot])
            pltpu.async_copy(cols.at[pl.ds(base, C)], cidx.at[pl.ds(ro, C)],
                             semi.at[islot])

        fire_idx(0, 0)

        if True:
            def chunk_body(cc, nb_prev):
                l, base = idx_base(cc)
                islot = cc % 2
                @pl.when(cc + 1 < L * CPT)
                def _():
                    fire_idx(cc + 1, 1 - islot)
                ro = pl.multiple_of(islot * C, 64)
                pltpu.make_async_copy(rows.at[pl.ds(0, C)],
                                     ridx.at[pl.ds(ro, C)],
                                     semi.at[islot]).wait()
                pltpu.make_async_copy(cols.at[pl.ds(0, C)],
                                     cidx.at[pl.ds(ro, C)],
                                     semi.at[islot]).wait()
                rid = ridx.at[pl.ds(ro, C)]
                cid = cidx.at[pl.ds(ro, C)]

                def fvreg(i, p):
                    rv = rid[pl.ds(16 * i, 16)]
                    cv = cid[pl.ds(16 * i, 16)]
                    loc = rv - lo
                    m = (loc >= 0) & (loc < Q)
                    mi = jnp.where(m, jnp.int32(1), jnp.int32(0))
                    cs = plsc.cumsum(mi)
                    pos = p + cs - mi
                    plsc.store_scatter(fridx, [pos], loc, mask=m)
                    plsc.store_scatter(fcidx, [pos], cv, mask=m)
                    return p + cs[15]

                p = lax.fori_loop(0, C // 16, fvreg, 0)
                iota = lax.iota(jnp.int32, 16)
                for j in range(G // 16):
                    fridx[pl.ds(p + 16 * j, 16)] = Q + 16 * j + iota
                    fcidx[pl.ds(p + 16 * j, 16)] = iota * 3000 + 750 * j + 7
                nb = (p + G - 1) // G

                def sidx_fill(b, _):
                    o = pl.multiple_of(b * G, G)
                    srow = sidx.at[b]
                    for j in range(G // 16):
                        srow[pl.ds(16 * j, 16)] = fridx[pl.ds(o + 16 * j, 16)]
                    return 0
                lax.fori_loop(0, nb, sidx_fill, 0)

                def fire_gather(b, slot):
                    o = pl.multiple_of(b * G, G)
                    pltpu.async_copy(
                        y.at[l].at[fcidx.at[pl.ds(o, G)]],
                        gbuf.at[slot], semg.at[slot],
                    )

                for pre in range(NSLOT):
                    @pl.when(pre < nb_prev)
                    def _():
                        pltpu.make_async_copy(
                            gbuf.at[pre], acc_sp.at[sidx.at[0]],
                            sems.at[pre]).wait()
                        pltpu.make_async_copy(
                            ones_g, deg_sp.at[sidx.at[0]],
                            sems.at[pre]).wait()
                    @pl.when(pre < nb)
                    def _():
                        fire_gather(pre, pre)

                def batch_body(b, _):
                    slot = b % NSLOT
                    gslot = gbuf.at[slot]
                    pltpu.make_async_copy(
                        y.at[l].at[fcidx.at[pl.ds(0, G)]], gslot,
                        semg.at[slot]).wait()
                    pltpu.async_copy(gslot, acc_sp.at[sidx.at[b]],
                                     sems.at[slot], add=True)
                    pltpu.async_copy(ones_g, deg_sp.at[sidx.at[b]],
                                     sems.at[slot], add=True)
                    @pl.when(b + NSLOT < nb)
                    def _():
                        pltpu.make_async_copy(
                            gslot, acc_sp.at[sidx.at[b]],
                            sems.at[slot]).wait()
                        pltpu.make_async_copy(
                            ones_g, deg_sp.at[sidx.at[b]],
                            sems.at[slot]).wait()
                        fire_gather(b + NSLOT, slot)
                    return 0
                lax.fori_loop(0, nb, batch_body, 0)
                return jnp.minimum(nb, NSLOT)
            nb_last = lax.fori_loop(0, L * CPT, chunk_body, 0)

            def tail_drain(e, _):
                @pl.when(e < nb_last)
                def _():
                    pltpu.make_async_copy(
                        gbuf.at[e], acc_sp.at[sidx.at[0]],
                        sems.at[e]).wait()
                    pltpu.make_async_copy(
                        ones_g, deg_sp.at[sidx.at[0]],
                        sems.at[e]).wait()
                return 0
            lax.fori_loop(0, NSLOT, tail_drain, 0)

        plsc.subcore_barrier()
        w0 = pl.multiple_of(s * Q + tr0, 16)
        last = (s == NQ - 1) & (t == 15)
        @pl.when(~last)
        def _():
            pltpu.sync_copy(acc_sp.at[pl.ds(tr0, RPT)],
                            out_u.at[pl.ds(w0, RPT)])
        @pl.when(last)
        def _():
            nl = N - (NQ - 1) * Q - 15 * RPT
            pltpu.sync_copy(acc_sp.at[pl.ds(tr0, nl)],
                            out_u.at[pl.ds(w0, nl)])
        pltpu.async_copy(deg_sp.at[zidx], dv, sem).wait()
        pltpu.sync_copy(dv, deg.at[pl.ds(w0, RPT)])
        plsc.subcore_barrier()


def _sc_stage(y, rows, cols):
    z2d = jnp.zeros((RPT, D), jnp.float32)
    f = pl.kernel(
        _sc_body,
        out_type=[
            jax.ShapeDtypeStruct((N, D), jnp.float32),
            jax.ShapeDtypeStruct((NQ * Q,), jnp.float32),
        ],
        mesh=plsc.VectorSubcoreMesh(core_axis_name="c", subcore_axis_name="s"),
        scratch_types=[
            pltpu.VMEM_SHARED((QP, D), jnp.float32),
            pltpu.VMEM_SHARED((QP,), jnp.float32),
            pltpu.VMEM((RPT,), jnp.int32),
            pltpu.VMEM((RPT,), jnp.float32),
            pltpu.VMEM((RPT,), jnp.float32),
            pltpu.VMEM((2 * C,), jnp.int32),
            pltpu.VMEM((2 * C,), jnp.int32),
            pltpu.VMEM((CP,), jnp.int32),
            pltpu.VMEM((CP,), jnp.int32),
            pltpu.VMEM((NBMAX, G), jnp.int32),
            pltpu.VMEM((NSLOT, G, D), jnp.float32),
            pltpu.VMEM((G,), jnp.float32),
            pltpu.SemaphoreType.DMA,
            pltpu.SemaphoreType.DMA((NSLOT,)),
            pltpu.SemaphoreType.DMA((NSLOT,)),
            pltpu.SemaphoreType.DMA((2,)),
        ],
        compiler_params=pltpu.CompilerParams(needs_layout_passes=False),
    )
    return f(y, rows, cols, z2d)


def kernel(features, adjacencies, adjacencies_t, w, bias, w_t, bias_t):
    w_all = jnp.concatenate([w, w_t], axis=0)
    b_all = jnp.concatenate([bias, bias_t], axis=0)

    pad_r = jnp.broadcast_to(jnp.int32(2 * N), (L, EP - E))
    pad_c = jnp.broadcast_to(
        (jnp.arange(EP - E, dtype=jnp.int32) * 977) % N, (L, EP - E))
    rows = jnp.concatenate(
        [adjacencies[:, 0, :], adjacencies_t[:, 0, :]], axis=0)
    cols = jnp.concatenate(
        [adjacencies[:, 1, :], adjacencies_t[:, 1, :]], axis=0)
    rows_p = jnp.concatenate([rows, pad_r], axis=1).reshape(L * EP)
    cols_p = jnp.concatenate([cols, pad_c], axis=1).reshape(L * EP)

    y = _mm_stage(features, w_all, b_all)
    out_u, deg = _sc_stage(y, rows_p, cols_p)
    return _norm_stage(out_u, deg)
